# baseline (speedup 1.0000x reference)
import jax
import jax.numpy as jnp
from jax.experimental import pallas as pl
from jax.experimental.pallas import tpu as pltpu

NUM_ENVS = 4096
N_IN = 12
NNZ_OUT = 8


def _compute(w):
    w0, w1, w2, w3, w4, w5, w6, w7, w8, w9, w10, w11 = w
    c12 = 0.5
    c13 = 2.0
    t14 = w0 + w1
    t15 = w2 * w3
    t14 = t14 - t15
    t15 = jnp.sin(w4)
    n0 = jnp.cos(w5)
    n1 = t15 * n0
    n2 = t14 + n1
    n3 = w6 * w6
    n4 = w7 * w7
    n3 = n3 + n4
    n3 = jnp.sqrt(n3)
    n3 = n3 + c12
    n4 = n2 / n3
    n5 = -w8
    n5 = n5 * c12
    n6 = w9 + c13
    n7 = w10 * w11
    n8 = jnp.tan(n4)
    n9 = n5 + n6
    n10 = n7 - n8
    n11 = jnp.sin(n9)
    n12 = jnp.cos(n10)
    n13 = n11 * n12
    t14 = n2 + n13
    t15 = n3 * n3
    return [t14, n4, n9, n10, n11, n12, n13, t15]


def _body(x_hbm, o_ref, xv, sem):
    pltpu.async_copy(x_hbm, xv, sem).wait()
    w = [xv[j] for j in range(N_IN)]
    outs = _compute(w)
    o_ref[...] = jnp.stack(outs, axis=0)


def kernel(input_batch):
    x = jax.lax.transpose(input_batch, (1, 2, 0)).reshape(N_IN, NUM_ENVS // 128, 128)
    out = pl.pallas_call(
        _body,
        in_specs=[pl.BlockSpec(memory_space=pl.ANY)],
        out_shape=jax.ShapeDtypeStruct((NNZ_OUT, NUM_ENVS // 128, 128), jnp.float32),
        scratch_shapes=[
            pltpu.VMEM((N_IN, NUM_ENVS // 128, 128), jnp.float32),
            pltpu.SemaphoreType.DMA,
        ],
    )(x)
    return jax.lax.transpose(out.reshape(NNZ_OUT, 1, NUM_ENVS), (2, 0, 1))
